# baseline (speedup 1.0000x reference)
import functools
import math

import jax
import jax.numpy as jnp
from jax import lax
from jax.experimental import pallas as pl
from jax.experimental.pallas import tpu as pltpu
from jax.experimental.pallas import tpu_sc as plsc


def _angular(l):
    return [(lx, ly, l - lx - ly)
            for lx in range(l, -1, -1)
            for ly in range(l - lx, -1, -1)]


_MAXL = 3
_LVECS = [v for l in range(_MAXL + 1) for v in _angular(l)]
_LIDX = {v: i for i, v in enumerate(_LVECS)}


def _mult(v):
    l = v[0] + v[1] + v[2]
    return math.factorial(l) // (
        math.factorial(v[0]) * math.factorial(v[1]) * math.factorial(v[2]))


_NL = len(_LVECS)
_NSYM = 6
_NCH = 8
_IN_ROW = _NL * _NCH
_OUT_ROW = _NSYM * _NCH
_GF = 16
_NW = 32

_L1 = _angular(1)
_L2 = _angular(2)
_L3 = _angular(3)


def _compute_group(in_ref, out_ref, base_in, base_out):
    for c in range(_NCH):
        x = [plsc.load_gather(in_ref, [base_in + (li * _NCH + c)])
             for li in range(_NL)]
        x2p = {v: (x[_LIDX[v]] if _mult(v) == 1 else x[_LIDX[v]] * float(_mult(v)))
               for v in _L2}
        x3p = {v: (x[_LIDX[v]] if _mult(v) == 1 else x[_LIDX[v]] * float(_mult(v)))
               for v in _L3}

        s1 = functools.reduce(
            lambda a, b: a + b, [x[_LIDX[v]] * x[_LIDX[v]] for v in _L1])
        s2 = functools.reduce(
            lambda a, b: a + b, [x[_LIDX[v]] * x2p[v] for v in _L2])
        s3 = functools.reduce(
            lambda a, b: a + b, [x[_LIDX[v]] * x3p[v] for v in _L3])
        t4 = []
        for i in range(3):
            for j in range(i, 3):
                v3 = tuple(p + q for p, q in zip(_L1[i], _L1[j]))
                t4.append(x[_LIDX[_L1[i]]] * x[_LIDX[_L1[j]]] * x2p[v3])
        s4 = functools.reduce(lambda a, b: a + b, t4)
        t5 = []
        for v1 in _L1:
            for v2 in _L2:
                v3 = tuple(p + q for p, q in zip(v1, v2))
                t5.append(x[_LIDX[v1]] * x2p[v2] * x[_LIDX[v3]])
        s5 = functools.reduce(lambda a, b: a + b, t5)

        for s, val in enumerate((x[0], s1, s2, s3, s4, s5)):
            plsc.store_scatter(out_ref, [base_out + (s * _NCH + c)], val)


def _sym_body(x_hbm, out_hbm, in_buf, out_buf):
    wid = lax.axis_index("s") * 2 + lax.axis_index("c")
    ngroups = x_hbm.shape[0] // (_GF * _IN_ROW)
    my_n = (ngroups - 1 - wid) // _NW + 1
    iota = lax.iota(jnp.int32, _GF)
    base_in = iota * _IN_ROW
    base_out = iota * _OUT_ROW
    in_slab = _GF * _IN_ROW
    out_slab = _GF * _OUT_ROW

    def body(i, carry):
        g = wid + i * _NW
        pltpu.sync_copy(x_hbm.at[pl.ds(g * in_slab, in_slab)], in_buf)
        _compute_group(in_buf, out_buf, base_in, base_out)
        pltpu.sync_copy(out_buf, out_hbm.at[pl.ds(g * out_slab, out_slab)])
        return carry

    lax.fori_loop(0, my_n, body, 0)


def kernel(node_attr):
    n, r, nl, ch = node_attr.shape
    assert nl == _NL and ch == _NCH and (n * r) % _GF == 0
    x = node_attr.reshape(n * r * nl * ch)
    mesh = plsc.VectorSubcoreMesh(core_axis_name="c", subcore_axis_name="s")
    out = pl.kernel(
        _sym_body,
        out_type=jax.ShapeDtypeStruct((n * r * _OUT_ROW,), jnp.float32),
        mesh=mesh,
        compiler_params=pltpu.CompilerParams(needs_layout_passes=False),
        scratch_types=[
            pltpu.VMEM((_GF * _IN_ROW,), jnp.float32),
            pltpu.VMEM((_GF * _OUT_ROW,), jnp.float32),
        ],
    )(x)
    return out.reshape(n, r, _NSYM, ch)
